# baseline (speedup 1.0000x reference)
import functools

import jax
import jax.numpy as jnp
from jax import lax
from jax.experimental import pallas as pl
from jax.experimental.pallas import tpu as pltpu
from jax.experimental.pallas import tpu_sc as plsc

NP_, NG_ = 8000, 2000
N_ = NP_ + NG_
HID = 128
NCLS = 500

NC, NS = 2, 16
NTILES = NC * NS
CHUNK = 128
SUB = 8
SCHUNK = SUB * CHUNK
E_REAL = 200000 + 60000 + 60000
SCPT = -(-E_REAL // (NTILES * SCHUNK))
E_PAD = NTILES * SCPT * SCHUNK
GR = E_PAD // SCHUNK
SCPT0, SCPT1 = 12, 8
PER_PAIR = SCPT0 + SCPT1
MAXSC = max(SCPT0, SCPT1)

NT = 10368
DUMMY = 10200
CNT = 18432
DUMMY_CNT = 18431
CROWS = CNT // NS
NCH = NT // CHUNK

BR = 400
BR2 = 648

_NEG = -1e30


def _vtake(v, idx):
    return v.at[idx].get(mode="promise_in_bounds")


_IOTA = functools.partial(lax.iota, jnp.int32)


def _seg_combine(keys, vals, op):
    idx = _IOTA(16)
    for step in (1, 2, 4, 8):
        sh = jnp.maximum(idx - step, 0)
        k2 = _vtake(keys, sh)
        v2 = _vtake(vals, sh)
        vals = jnp.where((k2 == keys) & (idx >= step), op(vals, v2), vals)
    nxt = _vtake(keys, jnp.minimum(idx + 1, 15))
    is_last = (nxt != keys) | (idx == 15)
    return vals, is_last


def _mesh():
    return plsc.VectorSubcoreMesh(core_axis_name="c", subcore_axis_name="s")


_SC_PARAMS = dict(
    mesh=None,
)


def _sc_count(cidx3d):
    @functools.partial(
        pl.kernel,
        out_type=jax.ShapeDtypeStruct((NTILES, 1, CNT), jnp.float32),
        mesh=_mesh(),
        compiler_params=pltpu.CompilerParams(needs_layout_passes=False),
        scratch_types=[
            pltpu.VMEM((SUB, CHUNK), jnp.int32),
            pltpu.VMEM((CNT,), jnp.float32),
        ],
    )
    def k(cidx_hbm, outp, idxv, ctab):
        cid = lax.axis_index("c")
        sid = lax.axis_index("s")
        wid = cid * NS + sid

        def cinit(i, _):
            ctab[pl.ds(i * 16, 16)] = jnp.zeros((16,), jnp.float32)
            return 0

        lax.fori_loop(0, CNT // 16, cinit, 0)

        def chunk(c, _):
            g = wid * SCPT + c
            pltpu.sync_copy(cidx_hbm.at[g], idxv)

            def group(gg, _):
                r = gg // (CHUNK // 16)
                base = (gg % (CHUNK // 16)) * 16
                idx16 = idxv[r, pl.ds(base, 16)]
                ks, vs = plsc.sort_key_val(idx16, jnp.ones((16,), jnp.float32))
                vs2, is_last = _seg_combine(ks, vs, lambda a, b: a + b)
                cur = plsc.load_gather(ctab, [ks])
                plsc.store_scatter(ctab, [ks], cur + vs2, mask=is_last)
                return 0

            lax.fori_loop(0, SCHUNK // 16, group, 0)
            return 0

        lax.fori_loop(0, SCPT, chunk, 0)
        pltpu.sync_copy(ctab, outp.at[wid, 0])

    return k(cidx3d)


def _sc_score(u, v, src3d, dst3d, cidx3d, inv):
    @functools.partial(
        pl.kernel,
        out_type=(jax.ShapeDtypeStruct((GR, SUB, CHUNK), jnp.float32),
                  jax.ShapeDtypeStruct((NTILES, 1, NT), jnp.float32),
                  jax.ShapeDtypeStruct((NTILES, 1, NT), jnp.float32)),
        mesh=_mesh(),
        compiler_params=pltpu.CompilerParams(needs_layout_passes=False),
        scratch_types=[
            pltpu.VMEM((CNT,), jnp.float32),
            pltpu.VMEM((NT,), jnp.float32),
            pltpu.VMEM((NT,), jnp.float32),
            pltpu.VMEM((MAXSC * SCHUNK,), jnp.float32),
            pltpu.VMEM((SUB, CHUNK), jnp.int32),
            pltpu.VMEM((SUB, CHUNK), jnp.int32),
            pltpu.VMEM((SUB, CHUNK), jnp.int32),
            pltpu.VMEM((CHUNK, HID), jnp.float32),
            pltpu.VMEM((CHUNK, HID), jnp.float32),
            pltpu.VMEM((CHUNK, HID), jnp.float32),
            pltpu.VMEM((CHUNK, HID), jnp.float32),
            pltpu.VMEM((SUB, CHUNK), jnp.float32),
            pltpu.VMEM((256,), jnp.float32),
            pltpu.SemaphoreType.DMA,
            pltpu.SemaphoreType.DMA,
            pltpu.SemaphoreType.DMA,
            pltpu.SemaphoreType.DMA,
        ],
    )
    def k(u_hbm, v_hbm, src_hbm, dst_hbm, cidx_hbm, inv_hbm,
          w_hbm, mloc_hbm, sloc_hbm,
          invtab, mtab, stab, elocal, srcv, dstv, cidxv,
          ub0, ub1, vb0, vb1, wv, trans, su0, su1, sv0, sv1):
        cid = lax.axis_index("c")
        sid = lax.axis_index("s")
        wid = cid * NS + sid
        my_n = jnp.where(cid == 0, SCPT0, SCPT1)
        gbase = sid * PER_PAIR + cid * SCPT0
        ub = (ub0, ub1)
        vb = (vb0, vb1)
        su = (su0, su1)
        sv = (sv0, sv1)
        pltpu.sync_copy(inv_hbm, invtab)

        def minit(i, _):
            mtab[pl.ds(i * 16, 16)] = jnp.full((16,), _NEG, jnp.float32)
            stab[pl.ds(i * 16, 16)] = jnp.zeros((16,), jnp.float32)
            return 0

        lax.fori_loop(0, NT // 16, minit, 0)

        def chunk(c, _):
            g = gbase + c
            pltpu.sync_copy(src_hbm.at[g], srcv)
            pltpu.sync_copy(dst_hbm.at[g], dstv)
            pltpu.sync_copy(cidx_hbm.at[g], cidxv)

            prev = (pltpu.async_copy(u_hbm.at[srcv.at[0]], ub[0], su[0]),
                    pltpu.async_copy(v_hbm.at[dstv.at[0]], vb[0], sv[0]))
            for r in range(SUB):
                if r < SUB - 1:
                    p = (r + 1) % 2
                    nxt = (pltpu.async_copy(u_hbm.at[srcv.at[r + 1]], ub[p], su[p]),
                           pltpu.async_copy(v_hbm.at[dstv.at[r + 1]], vb[p], sv[p]))
                prev[0].wait()
                prev[1].wait()
                urows = ub[r % 2]
                vrows = vb[r % 2]

                def group(gi, _):
                    base = gi * 16
                    for j in range(16):
                        acc = None
                        for kk in range(8):
                            z = (urows[base + j, pl.ds(kk * 16, 16)]
                                 + vrows[base + j, pl.ds(kk * 16, 16)])
                            t = jnp.maximum(z, 0.2 * z)
                            acc = t if kk == 0 else acc + t
                        trans[pl.ds(j * 16, 16)] = acc
                    iota16 = _IOTA(16) * 16
                    tot = None
                    for ll in range(16):
                        col = plsc.load_gather(trans, [iota16 + ll])
                        tot = col if ll == 0 else tot + col
                    cidx16 = cidxv[r, pl.ds(base, 16)]
                    alpha16 = plsc.load_gather(invtab, [cidx16])
                    e16 = tot * alpha16
                    elocal[pl.ds(c * SCHUNK + r * CHUNK + base, 16)] = e16
                    dst16 = dstv[r, pl.ds(base, 16)]
                    ks, vs = plsc.sort_key_val(dst16, e16)
                    vs2, is_last = _seg_combine(ks, vs, jnp.maximum)
                    cur = plsc.load_gather(mtab, [ks])
                    plsc.store_scatter(mtab, [ks], jnp.maximum(cur, vs2),
                                       mask=is_last)
                    return 0

                lax.fori_loop(0, CHUNK // 16, group, 0)
                if r < SUB - 1:
                    prev = nxt
            return 0

        lax.fori_loop(0, my_n, chunk, 0)
        pltpu.sync_copy(mtab, mloc_hbm.at[wid, 0])

        def chunk2(c, _):
            g = gbase + c
            pltpu.sync_copy(dst_hbm.at[g], dstv)

            def group(gg, _):
                r = gg // (CHUNK // 16)
                base = (gg % (CHUNK // 16)) * 16
                dst16 = dstv[r, pl.ds(base, 16)]
                e16 = elocal[pl.ds(c * SCHUNK + r * CHUNK + base, 16)]
                mg = plsc.load_gather(mtab, [dst16])
                w16 = jnp.exp(e16 - mg)
                wv[r, pl.ds(base, 16)] = w16
                ks, vs = plsc.sort_key_val(dst16, w16)
                vs2, is_last = _seg_combine(ks, vs, lambda a, b: a + b)
                cur = plsc.load_gather(stab, [ks])
                plsc.store_scatter(stab, [ks], cur + vs2, mask=is_last)
                return 0

            lax.fori_loop(0, SCHUNK // 16, group, 0)
            pltpu.sync_copy(wv, w_hbm.at[g])
            return 0

        lax.fori_loop(0, my_n, chunk2, 0)
        pltpu.sync_copy(stab, sloc_hbm.at[wid, 0])

    return k(u, v, src3d, dst3d, cidx3d, inv)


def _sc_scatter(x, src3d, dst3d, w3d, fac):
    CPW = -(-NCH // NS)

    @functools.partial(
        pl.kernel,
        out_type=jax.ShapeDtypeStruct((NC, NT, HID), jnp.float32),
        mesh=_mesh(),
        compiler_params=pltpu.CompilerParams(needs_layout_passes=False),
        scratch_types=[
            pltpu.VMEM((NT,), jnp.float32),
            pltpu.VMEM((SUB, CHUNK), jnp.int32),
            pltpu.VMEM((SUB, CHUNK), jnp.int32),
            pltpu.VMEM((SUB, CHUNK), jnp.float32),
            pltpu.VMEM((CHUNK, HID), jnp.float32),
            pltpu.VMEM((CHUNK, HID), jnp.float32),
            pltpu.VMEM_SHARED((NT, HID), jnp.float32),
            pltpu.SemaphoreType.DMA,
            pltpu.SemaphoreType.DMA,
        ],
    )
    def k(x_hbm, src_hbm, dst_hbm, w_hbm, f_hbm, outp,
          ftab, srcv, dstv, wv, xb0, xb1, accum, s0, s1):
        cid = lax.axis_index("c")
        sid = lax.axis_index("s")
        wid = cid * NS + sid
        my_n = jnp.where(cid == 0, SCPT0, SCPT1)
        gbase = sid * PER_PAIR + cid * SCPT0
        xb = (xb0, xb1)
        sems = (s0, s1)
        pltpu.sync_copy(f_hbm.at[wid, 0], ftab)

        def zfill(i, _):
            xb0[i // 8, pl.ds((i % 8) * 16, 16)] = jnp.zeros((16,), jnp.float32)
            return 0

        lax.fori_loop(0, CHUNK * 8, zfill, 0)

        def zslice(rr, _):
            ch = sid * CPW + rr

            @pl.when(ch < NCH)
            def _():
                pltpu.sync_copy(xb0, accum.at[pl.ds(ch * CHUNK, CHUNK)])

            return 0

        lax.fori_loop(0, CPW, zslice, 0)
        plsc.subcore_barrier()

        def chunk(c, _):
            g = gbase + c
            pltpu.sync_copy(src_hbm.at[g], srcv)
            pltpu.sync_copy(dst_hbm.at[g], dstv)
            pltpu.sync_copy(w_hbm.at[g], wv)

            prev = pltpu.async_copy(x_hbm.at[srcv.at[0]], xb[0], sems[0])
            for r in range(SUB):
                if r < SUB - 1:
                    p = (r + 1) % 2
                    nxt = pltpu.async_copy(x_hbm.at[srcv.at[r + 1]], xb[p], sems[p])
                prev.wait()
                xrows = xb[r % 2]

                def group(gi, _):
                    base = gi * 16
                    dst16 = dstv[r, pl.ds(base, 16)]
                    w16 = wv[r, pl.ds(base, 16)]
                    a16 = w16 * plsc.load_gather(ftab, [dst16])
                    for j in range(16):
                        aj = _vtake(a16, jnp.full((16,), j, jnp.int32))
                        for kk in range(8):
                            xrows[base + j, pl.ds(kk * 16, 16)] = (
                                xrows[base + j, pl.ds(kk * 16, 16)] * aj)
                    return 0

                lax.fori_loop(0, CHUNK // 16, group, 0)
                pltpu.sync_copy(xrows, accum.at[dstv.at[r]], add=True)
                if r < SUB - 1:
                    prev = nxt
            return 0

        lax.fori_loop(0, my_n, chunk, 0)
        plsc.subcore_barrier()

        def dump(rr, _):
            ch = sid * CPW + rr

            @pl.when(ch < NCH)
            def _():
                pltpu.sync_copy(accum.at[pl.ds(ch * CHUNK, CHUNK)], xb0)
                pltpu.sync_copy(xb0, outp.at[cid, pl.ds(ch * CHUNK, CHUNK)])

            return 0

        lax.fori_loop(0, CPW, dump, 0)

    return k(x, src3d, dst3d, w3d, fac)


def _ln_block(x, g, b):
    mu = jnp.mean(x, axis=-1, keepdims=True)
    va = jnp.var(x, axis=-1, keepdims=True)
    return (x - mu) / jnp.sqrt(va + 1e-5) * g + b


def _tc_fuse(xm, esm, wm, bm, we, lg, lb, nrows):
    def body(xm_ref, esm_ref, wm_ref, bm_ref, we_ref, lg_ref, lb_ref, o_ref):
        h = jnp.concatenate([xm_ref[...] @ wm_ref[...] + bm_ref[...],
                             esm_ref[...] @ we_ref[...]], axis=1)
        o_ref[...] = _ln_block(h, lg_ref[...], lb_ref[...])

    full = lambda shape: pl.BlockSpec(shape, lambda i: tuple(0 for _ in shape))
    return pl.pallas_call(
        body,
        grid=(nrows // BR,),
        in_specs=[
            pl.BlockSpec((BR, 256), lambda i: (i, 0)),
            pl.BlockSpec((BR, 1280), lambda i: (i, 0)),
            full((256, 64)), full((64,)), full((1280, 64)),
            full((HID,)), full((HID,)),
        ],
        out_specs=pl.BlockSpec((BR, HID), lambda i: (i, 0)),
        out_shape=jax.ShapeDtypeStruct((nrows, HID), jnp.float32),
    )(xm, esm, wm, bm, we, lg, lb)


def _tc_uv(x, wl, wr):
    def body(x_ref, wl_ref, wr_ref, u_ref, v_ref):
        xb = x_ref[...]
        u_ref[...] = xb @ wl_ref[...]
        v_ref[...] = xb @ wr_ref[...]

    w = pl.BlockSpec((HID, HID), lambda i: (0, 0))
    return pl.pallas_call(
        body,
        grid=(NT // BR2,),
        in_specs=[pl.BlockSpec((BR2, HID), lambda i: (i, 0)), w, w],
        out_specs=[pl.BlockSpec((BR2, HID), lambda i: (i, 0))] * 2,
        out_shape=[jax.ShapeDtypeStruct((NT, HID), jnp.float32)] * 2,
    )(x, wl, wr)


def _tc_combine_uv(parts, wl, wr):
    def body(p_ref, wl_ref, wr_ref, x_ref, u_ref, v_ref):
        xb = p_ref[0] + p_ref[1]
        x_ref[...] = xb
        u_ref[...] = xb @ wl_ref[...]
        v_ref[...] = xb @ wr_ref[...]

    w = pl.BlockSpec((HID, HID), lambda i: (0, 0))
    return pl.pallas_call(
        body,
        grid=(NT // BR2,),
        in_specs=[pl.BlockSpec((NC, BR2, HID), lambda i: (0, i, 0)), w, w],
        out_specs=[pl.BlockSpec((BR2, HID), lambda i: (i, 0))] * 3,
        out_shape=[jax.ShapeDtypeStruct((NT, HID), jnp.float32)] * 3,
    )(parts, wl, wr)


def _tc_invcnt(parts):
    def body(p_ref, o_ref):
        cnt = jnp.sum(p_ref[...], axis=(0, 1))
        o_ref[...] = 1.0 / jnp.maximum(cnt, 0.5)

    return pl.pallas_call(
        body,
        grid=(1,),
        in_specs=[pl.BlockSpec((NTILES, 1, CNT), lambda i: (0, 0, 0))],
        out_specs=pl.BlockSpec((CNT,), lambda i: (0,)),
        out_shape=jax.ShapeDtypeStruct((CNT,), jnp.float32),
    )(parts)


def _tc_factors(mloc, sloc):
    def body(m_ref, s_ref, o_ref):
        ml = m_ref[...]
        m = jnp.max(ml, axis=0, keepdims=True)
        em = jnp.exp(ml - m)
        S = jnp.sum(s_ref[...] * em, axis=0, keepdims=True)
        o_ref[...] = em / jnp.maximum(S, 1e-12)

    return pl.pallas_call(
        body,
        grid=(1,),
        in_specs=[pl.BlockSpec((NTILES, 1, NT), lambda i: (0, 0, 0))] * 2,
        out_specs=pl.BlockSpec((NTILES, 1, NT), lambda i: (0, 0, 0)),
        out_shape=jax.ShapeDtypeStruct((NTILES, 1, NT), jnp.float32),
    )(mloc, sloc)


def _tc_classifier(parts, g, b, wc, bc):
    def body(p_ref, g_ref, b_ref, wc_ref, bc_ref, o_ref):
        xb = p_ref[0] + p_ref[1]
        h = _ln_block(xb, g_ref[...], b_ref[...])
        o_ref[...] = h @ wc_ref[...] + bc_ref[...]

    return pl.pallas_call(
        body,
        grid=(NP_ // BR,),
        in_specs=[
            pl.BlockSpec((NC, BR, HID), lambda i: (0, i, 0)),
            pl.BlockSpec((HID,), lambda i: (0,)),
            pl.BlockSpec((HID,), lambda i: (0,)),
            pl.BlockSpec((HID, NCLS), lambda i: (0, 0)),
            pl.BlockSpec((NCLS,), lambda i: (0,)),
        ],
        out_specs=pl.BlockSpec((BR, NCLS), lambda i: (i, 0)),
        out_shape=jax.ShapeDtypeStruct((NP_, NCLS), jnp.float32),
    )(parts, g, b, wc, bc)


def kernel(x_msa_protein, x_msa_go, esm2_protein, esm2_go, params,
           src_pp, dst_pp, src_pg, dst_pg, src_gp, dst_gp):
    P = params
    pad = E_PAD - E_REAL

    src_h = jnp.concatenate([src_pp, src_pg, src_gp + NP_,
                             jnp.full((pad,), DUMMY, jnp.int32)]
                            ).reshape(GR, SUB, CHUNK)
    dst_h = jnp.concatenate([dst_pp, dst_pg + NP_, dst_gp,
                             jnp.full((pad,), DUMMY, jnp.int32)]
                            ).reshape(GR, SUB, CHUNK)
    cidx = jnp.concatenate([dst_pp, dst_pg + NP_, dst_gp + N_,
                            jnp.full((pad,), DUMMY_CNT, jnp.int32)]
                           ).reshape(GR, SUB, CHUNK)

    cnt_parts = _sc_count(cidx)
    inv = _tc_invcnt(cnt_parts)

    hp = _tc_fuse(x_msa_protein, esm2_protein, P['W_msa_p'], P['b_msa_p'],
                  P['W_esm_p'], P['ln_p_g'], P['ln_p_b'], NP_)
    hg = _tc_fuse(x_msa_go, esm2_go, P['W_msa_g'], P['b_msa_g'],
                  P['W_esm_g'], P['ln_g_g'], P['ln_g_b'], NG_)
    x = jnp.concatenate([hp, hg, jnp.zeros((NT - N_, HID), jnp.float32)])

    for l in range(2):
        if l == 0:
            u, v = _tc_uv(x, P['na0_Wl'], P['na0_Wr'])
        else:
            x, u, v = _tc_combine_uv(parts, P['na1_Wl'], P['na1_Wr'])
        w3d, mloc, sloc = _sc_score(u, v, src_h, dst_h, cidx, inv)
        fac = _tc_factors(mloc, sloc)
        parts = _sc_scatter(x, src_h, dst_h, w3d, fac)

    return _tc_classifier(parts, P['hln_g'], P['hln_b'], P['W_cls'], P['b_cls'])


# speedup vs baseline: 1.1152x; 1.1152x over previous
import functools

import jax
import jax.numpy as jnp
from jax import lax
from jax.experimental import pallas as pl
from jax.experimental.pallas import tpu as pltpu
from jax.experimental.pallas import tpu_sc as plsc

NP_, NG_ = 8000, 2000
N_ = NP_ + NG_
HID = 128
NCLS = 500

NC, NS = 2, 16
NTILES = NC * NS
CHUNK = 128
SUB = 8
SCHUNK = SUB * CHUNK
E_REAL = 200000 + 60000 + 60000
SCPT = -(-E_REAL // (NTILES * SCHUNK))
E_PAD = NTILES * SCPT * SCHUNK
GR = E_PAD // SCHUNK
SCPT0, SCPT1 = 14, 6
PER_PAIR = SCPT0 + SCPT1
MAXSC = max(SCPT0, SCPT1)

NT = 10368
DUMMY = 10200
CNT = 18432
DUMMY_CNT = 18431
CROWS = CNT // NS
NCH = NT // CHUNK

BR = 400
BR2 = 648

_NEG = -1e30


def _vtake(v, idx):
    return v.at[idx].get(mode="promise_in_bounds")


_IOTA = functools.partial(lax.iota, jnp.int32)


def _seg_combine(keys, vals, op):
    idx = _IOTA(16)
    for step in (1, 2, 4, 8):
        sh = jnp.maximum(idx - step, 0)
        k2 = _vtake(keys, sh)
        v2 = _vtake(vals, sh)
        vals = jnp.where((k2 == keys) & (idx >= step), op(vals, v2), vals)
    nxt = _vtake(keys, jnp.minimum(idx + 1, 15))
    is_last = (nxt != keys) | (idx == 15)
    return vals, is_last


def _mesh():
    return plsc.VectorSubcoreMesh(core_axis_name="c", subcore_axis_name="s")


_SC_PARAMS = dict(
    mesh=None,
)


def _sc_count(cidx3d):
    @functools.partial(
        pl.kernel,
        out_type=jax.ShapeDtypeStruct((NTILES, 1, CNT), jnp.float32),
        mesh=_mesh(),
        compiler_params=pltpu.CompilerParams(needs_layout_passes=False),
        scratch_types=[
            pltpu.VMEM((SUB, CHUNK), jnp.int32),
            pltpu.VMEM((CNT,), jnp.float32),
        ],
    )
    def k(cidx_hbm, outp, idxv, ctab):
        cid = lax.axis_index("c")
        sid = lax.axis_index("s")
        wid = cid * NS + sid

        def cinit(i, _):
            ctab[pl.ds(i * 16, 16)] = jnp.zeros((16,), jnp.float32)
            return 0

        lax.fori_loop(0, CNT // 16, cinit, 0)

        def chunk(c, _):
            g = wid * SCPT + c
            pltpu.sync_copy(cidx_hbm.at[g], idxv)

            def group(gg, _):
                r = gg // (CHUNK // 16)
                base = (gg % (CHUNK // 16)) * 16
                idx16 = idxv[r, pl.ds(base, 16)]
                ks, vs = plsc.sort_key_val(idx16, jnp.ones((16,), jnp.float32))
                vs2, is_last = _seg_combine(ks, vs, lambda a, b: a + b)
                cur = plsc.load_gather(ctab, [ks])
                plsc.store_scatter(ctab, [ks], cur + vs2, mask=is_last)
                return 0

            lax.fori_loop(0, SCHUNK // 16, group, 0)
            return 0

        lax.fori_loop(0, SCPT, chunk, 0)
        pltpu.sync_copy(ctab, outp.at[wid, 0])

    return k(cidx3d)


def _sc_score(u, v, src3d, dst3d, cidx3d, inv):
    @functools.partial(
        pl.kernel,
        out_type=(jax.ShapeDtypeStruct((GR, SUB, CHUNK), jnp.float32),
                  jax.ShapeDtypeStruct((NTILES, 1, NT), jnp.float32),
                  jax.ShapeDtypeStruct((NTILES, 1, NT), jnp.float32)),
        mesh=_mesh(),
        compiler_params=pltpu.CompilerParams(needs_layout_passes=False),
        scratch_types=[
            pltpu.VMEM((CNT,), jnp.float32),
            pltpu.VMEM((NT,), jnp.float32),
            pltpu.VMEM((NT,), jnp.float32),
            pltpu.VMEM((MAXSC * SCHUNK,), jnp.float32),
            pltpu.VMEM((SUB, CHUNK), jnp.int32),
            pltpu.VMEM((SUB, CHUNK), jnp.int32),
            pltpu.VMEM((SUB, CHUNK), jnp.int32),
            pltpu.VMEM((CHUNK, HID), jnp.float32),
            pltpu.VMEM((CHUNK, HID), jnp.float32),
            pltpu.VMEM((CHUNK, HID), jnp.float32),
            pltpu.VMEM((CHUNK, HID), jnp.float32),
            pltpu.VMEM((SUB, CHUNK), jnp.float32),
            pltpu.VMEM((256,), jnp.float32),
            pltpu.SemaphoreType.DMA,
            pltpu.SemaphoreType.DMA,
            pltpu.SemaphoreType.DMA,
            pltpu.SemaphoreType.DMA,
        ],
    )
    def k(u_hbm, v_hbm, src_hbm, dst_hbm, cidx_hbm, inv_hbm,
          w_hbm, mloc_hbm, sloc_hbm,
          invtab, mtab, stab, elocal, srcv, dstv, cidxv,
          ub0, ub1, vb0, vb1, wv, trans, su0, su1, sv0, sv1):
        cid = lax.axis_index("c")
        sid = lax.axis_index("s")
        wid = cid * NS + sid
        my_n = jnp.where(cid == 0, SCPT0, SCPT1)
        gbase = sid * PER_PAIR + cid * SCPT0
        ub = (ub0, ub1)
        vb = (vb0, vb1)
        su = (su0, su1)
        sv = (sv0, sv1)
        pltpu.sync_copy(inv_hbm, invtab)

        def minit(i, _):
            mtab[pl.ds(i * 16, 16)] = jnp.full((16,), _NEG, jnp.float32)
            stab[pl.ds(i * 16, 16)] = jnp.zeros((16,), jnp.float32)
            return 0

        lax.fori_loop(0, NT // 16, minit, 0)

        def chunk(c, _):
            g = gbase + c
            pltpu.sync_copy(src_hbm.at[g], srcv)
            pltpu.sync_copy(dst_hbm.at[g], dstv)
            pltpu.sync_copy(cidx_hbm.at[g], cidxv)

            prev = (pltpu.async_copy(u_hbm.at[srcv.at[0]], ub[0], su[0]),
                    pltpu.async_copy(v_hbm.at[dstv.at[0]], vb[0], sv[0]))
            for r in range(SUB):
                if r < SUB - 1:
                    p = (r + 1) % 2
                    nxt = (pltpu.async_copy(u_hbm.at[srcv.at[r + 1]], ub[p], su[p]),
                           pltpu.async_copy(v_hbm.at[dstv.at[r + 1]], vb[p], sv[p]))
                prev[0].wait()
                prev[1].wait()
                urows = ub[r % 2]
                vrows = vb[r % 2]

                def group(gi, _):
                    base = gi * 16
                    for j in range(16):
                        acc = None
                        for kk in range(8):
                            z = (urows[base + j, pl.ds(kk * 16, 16)]
                                 + vrows[base + j, pl.ds(kk * 16, 16)])
                            t = jnp.maximum(z, 0.2 * z)
                            acc = t if kk == 0 else acc + t
                        trans[pl.ds(j * 16, 16)] = acc
                    iota16 = _IOTA(16) * 16
                    tot = None
                    for ll in range(16):
                        col = plsc.load_gather(trans, [iota16 + ll])
                        tot = col if ll == 0 else tot + col
                    cidx16 = cidxv[r, pl.ds(base, 16)]
                    alpha16 = plsc.load_gather(invtab, [cidx16])
                    e16 = tot * alpha16
                    elocal[pl.ds(c * SCHUNK + r * CHUNK + base, 16)] = e16
                    dst16 = dstv[r, pl.ds(base, 16)]
                    ks, vs = plsc.sort_key_val(dst16, e16)
                    vs2, is_last = _seg_combine(ks, vs, jnp.maximum)
                    cur = plsc.load_gather(mtab, [ks])
                    plsc.store_scatter(mtab, [ks], jnp.maximum(cur, vs2),
                                       mask=is_last)
                    return 0

                lax.fori_loop(0, CHUNK // 16, group, 0)
                if r < SUB - 1:
                    prev = nxt
            return 0

        lax.fori_loop(0, my_n, chunk, 0)
        pltpu.sync_copy(mtab, mloc_hbm.at[wid, 0])

        def chunk2(c, _):
            g = gbase + c
            pltpu.sync_copy(dst_hbm.at[g], dstv)

            def group(gg, _):
                r = gg // (CHUNK // 16)
                base = (gg % (CHUNK // 16)) * 16
                dst16 = dstv[r, pl.ds(base, 16)]
                e16 = elocal[pl.ds(c * SCHUNK + r * CHUNK + base, 16)]
                mg = plsc.load_gather(mtab, [dst16])
                w16 = jnp.exp(e16 - mg)
                wv[r, pl.ds(base, 16)] = w16
                ks, vs = plsc.sort_key_val(dst16, w16)
                vs2, is_last = _seg_combine(ks, vs, lambda a, b: a + b)
                cur = plsc.load_gather(stab, [ks])
                plsc.store_scatter(stab, [ks], cur + vs2, mask=is_last)
                return 0

            lax.fori_loop(0, SCHUNK // 16, group, 0)
            pltpu.sync_copy(wv, w_hbm.at[g])
            return 0

        lax.fori_loop(0, my_n, chunk2, 0)
        pltpu.sync_copy(stab, sloc_hbm.at[wid, 0])

    return k(u, v, src3d, dst3d, cidx3d, inv)


def _sc_scatter(x, src3d, dst3d, w3d, fac):
    CPW = -(-NCH // NS)

    @functools.partial(
        pl.kernel,
        out_type=jax.ShapeDtypeStruct((NC, NT, HID), jnp.float32),
        mesh=_mesh(),
        compiler_params=pltpu.CompilerParams(needs_layout_passes=False),
        scratch_types=[
            pltpu.VMEM((NT,), jnp.float32),
            pltpu.VMEM((SUB, CHUNK), jnp.int32),
            pltpu.VMEM((SUB, CHUNK), jnp.int32),
            pltpu.VMEM((SUB, CHUNK), jnp.float32),
            pltpu.VMEM((CHUNK, HID), jnp.float32),
            pltpu.VMEM((CHUNK, HID), jnp.float32),
            pltpu.VMEM_SHARED((NT, HID), jnp.float32),
            pltpu.SemaphoreType.DMA,
            pltpu.SemaphoreType.DMA,
        ],
    )
    def k(x_hbm, src_hbm, dst_hbm, w_hbm, f_hbm, outp,
          ftab, srcv, dstv, wv, xb0, xb1, accum, s0, s1):
        cid = lax.axis_index("c")
        sid = lax.axis_index("s")
        wid = cid * NS + sid
        my_n = jnp.where(cid == 0, SCPT0, SCPT1)
        gbase = sid * PER_PAIR + cid * SCPT0
        xb = (xb0, xb1)
        sems = (s0, s1)
        pltpu.sync_copy(f_hbm.at[wid, 0], ftab)

        def zfill(i, _):
            xb0[i // 8, pl.ds((i % 8) * 16, 16)] = jnp.zeros((16,), jnp.float32)
            return 0

        lax.fori_loop(0, CHUNK * 8, zfill, 0)

        def zslice(rr, _):
            ch = sid * CPW + rr

            @pl.when(ch < NCH)
            def _():
                pltpu.sync_copy(xb0, accum.at[pl.ds(ch * CHUNK, CHUNK)])

            return 0

        lax.fori_loop(0, CPW, zslice, 0)
        plsc.subcore_barrier()

        def chunk(c, _):
            g = gbase + c
            pltpu.sync_copy(src_hbm.at[g], srcv)
            pltpu.sync_copy(dst_hbm.at[g], dstv)
            pltpu.sync_copy(w_hbm.at[g], wv)

            prev = pltpu.async_copy(x_hbm.at[srcv.at[0]], xb[0], sems[0])
            for r in range(SUB):
                if r < SUB - 1:
                    p = (r + 1) % 2
                    nxt = pltpu.async_copy(x_hbm.at[srcv.at[r + 1]], xb[p], sems[p])
                prev.wait()
                xrows = xb[r % 2]

                def group(gi, _):
                    base = gi * 16
                    dst16 = dstv[r, pl.ds(base, 16)]
                    w16 = wv[r, pl.ds(base, 16)]
                    a16 = w16 * plsc.load_gather(ftab, [dst16])
                    for j in range(16):
                        aj = _vtake(a16, jnp.full((16,), j, jnp.int32))
                        for kk in range(8):
                            xrows[base + j, pl.ds(kk * 16, 16)] = (
                                xrows[base + j, pl.ds(kk * 16, 16)] * aj)
                    return 0

                lax.fori_loop(0, CHUNK // 16, group, 0)
                pltpu.sync_copy(xrows, accum.at[dstv.at[r]], add=True)
                if r < SUB - 1:
                    prev = nxt
            return 0

        lax.fori_loop(0, my_n, chunk, 0)
        plsc.subcore_barrier()

        def dump(rr, _):
            ch = sid * CPW + rr

            @pl.when(ch < NCH)
            def _():
                pltpu.sync_copy(accum.at[pl.ds(ch * CHUNK, CHUNK)], xb0)
                pltpu.sync_copy(xb0, outp.at[cid, pl.ds(ch * CHUNK, CHUNK)])

            return 0

        lax.fori_loop(0, CPW, dump, 0)

    return k(x, src3d, dst3d, w3d, fac)


def _ln_block(x, g, b):
    mu = jnp.mean(x, axis=-1, keepdims=True)
    va = jnp.var(x, axis=-1, keepdims=True)
    return (x - mu) / jnp.sqrt(va + 1e-5) * g + b


def _tc_fuse(xm, esm, wm, bm, we, lg, lb, nrows):
    def body(xm_ref, esm_ref, wm_ref, bm_ref, we_ref, lg_ref, lb_ref, o_ref):
        h = jnp.concatenate([xm_ref[...] @ wm_ref[...] + bm_ref[...],
                             esm_ref[...] @ we_ref[...]], axis=1)
        o_ref[...] = _ln_block(h, lg_ref[...], lb_ref[...])

    full = lambda shape: pl.BlockSpec(shape, lambda i: tuple(0 for _ in shape))
    return pl.pallas_call(
        body,
        grid=(nrows // BR,),
        in_specs=[
            pl.BlockSpec((BR, 256), lambda i: (i, 0)),
            pl.BlockSpec((BR, 1280), lambda i: (i, 0)),
            full((256, 64)), full((64,)), full((1280, 64)),
            full((HID,)), full((HID,)),
        ],
        out_specs=pl.BlockSpec((BR, HID), lambda i: (i, 0)),
        out_shape=jax.ShapeDtypeStruct((nrows, HID), jnp.float32),
    )(xm, esm, wm, bm, we, lg, lb)


def _tc_uv(x, wl, wr):
    def body(x_ref, wl_ref, wr_ref, u_ref, v_ref):
        xb = x_ref[...]
        u_ref[...] = xb @ wl_ref[...]
        v_ref[...] = xb @ wr_ref[...]

    w = pl.BlockSpec((HID, HID), lambda i: (0, 0))
    return pl.pallas_call(
        body,
        grid=(NT // BR2,),
        in_specs=[pl.BlockSpec((BR2, HID), lambda i: (i, 0)), w, w],
        out_specs=[pl.BlockSpec((BR2, HID), lambda i: (i, 0))] * 2,
        out_shape=[jax.ShapeDtypeStruct((NT, HID), jnp.float32)] * 2,
    )(x, wl, wr)


def _tc_combine_uv(parts, wl, wr):
    def body(p_ref, wl_ref, wr_ref, x_ref, u_ref, v_ref):
        xb = p_ref[0] + p_ref[1]
        x_ref[...] = xb
        u_ref[...] = xb @ wl_ref[...]
        v_ref[...] = xb @ wr_ref[...]

    w = pl.BlockSpec((HID, HID), lambda i: (0, 0))
    return pl.pallas_call(
        body,
        grid=(NT // BR2,),
        in_specs=[pl.BlockSpec((NC, BR2, HID), lambda i: (0, i, 0)), w, w],
        out_specs=[pl.BlockSpec((BR2, HID), lambda i: (i, 0))] * 3,
        out_shape=[jax.ShapeDtypeStruct((NT, HID), jnp.float32)] * 3,
    )(parts, wl, wr)


def _tc_invcnt(parts):
    def body(p_ref, o_ref):
        cnt = jnp.sum(p_ref[...], axis=(0, 1))
        o_ref[...] = 1.0 / jnp.maximum(cnt, 0.5)

    return pl.pallas_call(
        body,
        grid=(1,),
        in_specs=[pl.BlockSpec((NTILES, 1, CNT), lambda i: (0, 0, 0))],
        out_specs=pl.BlockSpec((CNT,), lambda i: (0,)),
        out_shape=jax.ShapeDtypeStruct((CNT,), jnp.float32),
    )(parts)


def _tc_factors(mloc, sloc):
    def body(m_ref, s_ref, o_ref):
        ml = m_ref[...]
        m = jnp.max(ml, axis=0, keepdims=True)
        em = jnp.exp(ml - m)
        S = jnp.sum(s_ref[...] * em, axis=0, keepdims=True)
        o_ref[...] = em / jnp.maximum(S, 1e-12)

    return pl.pallas_call(
        body,
        grid=(1,),
        in_specs=[pl.BlockSpec((NTILES, 1, NT), lambda i: (0, 0, 0))] * 2,
        out_specs=pl.BlockSpec((NTILES, 1, NT), lambda i: (0, 0, 0)),
        out_shape=jax.ShapeDtypeStruct((NTILES, 1, NT), jnp.float32),
    )(mloc, sloc)


def _tc_classifier(parts, g, b, wc, bc):
    def body(p_ref, g_ref, b_ref, wc_ref, bc_ref, o_ref):
        xb = p_ref[0] + p_ref[1]
        h = _ln_block(xb, g_ref[...], b_ref[...])
        o_ref[...] = h @ wc_ref[...] + bc_ref[...]

    return pl.pallas_call(
        body,
        grid=(NP_ // BR,),
        in_specs=[
            pl.BlockSpec((NC, BR, HID), lambda i: (0, i, 0)),
            pl.BlockSpec((HID,), lambda i: (0,)),
            pl.BlockSpec((HID,), lambda i: (0,)),
            pl.BlockSpec((HID, NCLS), lambda i: (0, 0)),
            pl.BlockSpec((NCLS,), lambda i: (0,)),
        ],
        out_specs=pl.BlockSpec((BR, NCLS), lambda i: (i, 0)),
        out_shape=jax.ShapeDtypeStruct((NP_, NCLS), jnp.float32),
    )(parts, g, b, wc, bc)


def kernel(x_msa_protein, x_msa_go, esm2_protein, esm2_go, params,
           src_pp, dst_pp, src_pg, dst_pg, src_gp, dst_gp):
    P = params
    pad = E_PAD - E_REAL

    src_h = jnp.concatenate([src_pp, src_pg, src_gp + NP_,
                             jnp.full((pad,), DUMMY, jnp.int32)]
                            ).reshape(GR, SUB, CHUNK)
    dst_h = jnp.concatenate([dst_pp, dst_pg + NP_, dst_gp,
                             jnp.full((pad,), DUMMY, jnp.int32)]
                            ).reshape(GR, SUB, CHUNK)
    cidx = jnp.concatenate([dst_pp, dst_pg + NP_, dst_gp + N_,
                            jnp.full((pad,), DUMMY_CNT, jnp.int32)]
                           ).reshape(GR, SUB, CHUNK)

    cnt_parts = _sc_count(cidx)
    inv = _tc_invcnt(cnt_parts)

    hp = _tc_fuse(x_msa_protein, esm2_protein, P['W_msa_p'], P['b_msa_p'],
                  P['W_esm_p'], P['ln_p_g'], P['ln_p_b'], NP_)
    hg = _tc_fuse(x_msa_go, esm2_go, P['W_msa_g'], P['b_msa_g'],
                  P['W_esm_g'], P['ln_g_g'], P['ln_g_b'], NG_)
    x = jnp.concatenate([hp, hg, jnp.zeros((NT - N_, HID), jnp.float32)])

    for l in range(2):
        if l == 0:
            u, v = _tc_uv(x, P['na0_Wl'], P['na0_Wr'])
        else:
            x, u, v = _tc_combine_uv(parts, P['na1_Wl'], P['na1_Wr'])
        w3d, mloc, sloc = _sc_score(u, v, src_h, dst_h, cidx, inv)
        fac = _tc_factors(mloc, sloc)
        parts = _sc_scatter(x, src_h, dst_h, w3d, fac)

    return _tc_classifier(parts, P['hln_g'], P['hln_b'], P['W_cls'], P['b_cls'])
